# baseline (speedup 1.0000x reference)
import functools

import jax
import jax.numpy as jnp
from jax import lax
from jax.experimental import pallas as pl
from jax.experimental.pallas import tpu as pltpu
from jax.experimental.pallas import tpu_sc as plsc

BN_EPS_ = 1e-5
_CH = 128
_TILES = 16


_NBUF = 2


def _edge_loop(x_hbm, src2d_hbm, dst2d_hbm, ibs, ibd, rows,
               isems, gsems, ssems, acc_sh, c0, nch):
    def idx_fetch(ci, slot):
        pltpu.async_copy(src2d_hbm.at[pl.ds(c0 + ci, 1)], ibs[slot],
                         isems[slot])
        pltpu.async_copy(dst2d_hbm.at[pl.ds(c0 + ci, 1)], ibd[slot],
                         isems[slot])

    def idx_wait(slot):
        pltpu.make_async_copy(
            src2d_hbm.at[pl.ds(c0, 1)], ibs[slot], isems[slot]).wait()
        pltpu.make_async_copy(
            dst2d_hbm.at[pl.ds(c0, 1)], ibd[slot], isems[slot]).wait()

    def scat_wait(k):
        pltpu.make_async_copy(rows[k], acc_sh.at[ibd[0].at[0]],
                              ssems[k]).wait()

    def sub_block(i, base, slots, first):
        for k in range(2):
            if first:
                @pl.when(i >= 4)
                def _():
                    scat_wait(k)
            else:
                scat_wait(k)
            idx_fetch(i + base + 2 + k, slots[k] ^ 2)
            idx_wait(slots[k])
            pltpu.async_copy(x_hbm.at[ibs[slots[k]].at[0]], rows[k],
                             gsems[k])
        for k in range(2):
            pltpu.make_async_copy(
                x_hbm.at[ibs[0].at[0]], rows[k], gsems[k]).wait()
            pltpu.async_copy(rows[k], acc_sh.at[ibd[slots[k]].at[0]],
                             ssems[k], add=True)

    @pl.loop(0, nch, step=4)
    def _(i):
        @pl.when(i == 0)
        def _():
            for k in range(2):
                idx_fetch(k, k)
        sub_block(i, 0, (0, 1), first=True)
        sub_block(i, 2, (2, 3), first=False)

    for k in range(2):
        scat_wait(k)
        idx_wait(k)


def _sc_mesh():
    return plsc.VectorSubcoreMesh(core_axis_name="c", subcore_axis_name="s")


def _agg_edge_split(x, zeros, src, dst):
    n, f = x.shape
    rpt = n // _TILES
    nch = (src.shape[0] - 2) // (2 * _TILES)

    @functools.partial(
        pl.kernel,
        out_type=(
            jax.ShapeDtypeStruct((n, f), jnp.float32),
            jax.ShapeDtypeStruct((n, f), jnp.float32),
        ),
        mesh=_sc_mesh(),
        scratch_types=(
            [pltpu.VMEM((1, _CH), jnp.int32)] * 8
            + [pltpu.VMEM((_CH, f), jnp.float32)] * 2
            + [pltpu.SemaphoreType.DMA] * 8
            + [pltpu.VMEM_SHARED((n, f), jnp.float32)]
        ),
    )
    def agg_kernel(x_hbm, z_hbm, src_hbm, dst_hbm, o0_hbm, o1_hbm, *rest):
        ibs, ibd, rows = rest[:4], rest[4:8], rest[8:10]
        isems, gsems, ssems = rest[10:14], rest[14:16], rest[16:18]
        acc_sh = rest[18]
        c = lax.axis_index("c")
        s = lax.axis_index("s")

        def run(init_hbm, o_hbm):
            pltpu.sync_copy(init_hbm.at[pl.ds(s * rpt, rpt)],
                            acc_sh.at[pl.ds(s * rpt, rpt)])
            plsc.subcore_barrier()
            _edge_loop(x_hbm, src_hbm, dst_hbm, ibs, ibd, rows, isems,
                       gsems, ssems, acc_sh, (c * _TILES + s) * nch, nch)
            plsc.subcore_barrier()
            pltpu.sync_copy(acc_sh.at[pl.ds(s * rpt, rpt)],
                            o_hbm.at[pl.ds(s * rpt, rpt)])

        @pl.when(c == 0)
        def _():
            run(x_hbm, o0_hbm)

        @pl.when(c == 1)
        def _():
            run(z_hbm, o1_hbm)

    return agg_kernel(x, zeros, src, dst)


def _agg_feat_split(x_lo, x_hi, src, dst):
    n, fh = x_lo.shape
    rpt = n // _TILES
    nch = (src.shape[0] - 2) // _TILES

    @functools.partial(
        pl.kernel,
        out_type=(
            jax.ShapeDtypeStruct((n, fh), jnp.float32),
            jax.ShapeDtypeStruct((n, fh), jnp.float32),
        ),
        mesh=_sc_mesh(),
        scratch_types=(
            [pltpu.VMEM((1, _CH), jnp.int32)] * 8
            + [pltpu.VMEM((_CH, fh), jnp.float32)] * 2
            + [pltpu.SemaphoreType.DMA] * 8
            + [pltpu.VMEM_SHARED((n, fh), jnp.float32)]
        ),
    )
    def agg_kernel(xlo_hbm, xhi_hbm, src_hbm, dst_hbm, olo_hbm, ohi_hbm,
                   *rest):
        ibs, ibd, rows = rest[:4], rest[4:8], rest[8:10]
        isems, gsems, ssems = rest[10:14], rest[14:16], rest[16:18]
        acc_sh = rest[18]
        c = lax.axis_index("c")
        s = lax.axis_index("s")

        def run(x_hbm, o_hbm):
            pltpu.sync_copy(x_hbm.at[pl.ds(s * rpt, rpt)],
                            acc_sh.at[pl.ds(s * rpt, rpt)])
            plsc.subcore_barrier()
            _edge_loop(x_hbm, src_hbm, dst_hbm, ibs, ibd, rows, isems,
                       gsems, ssems, acc_sh, s * nch, nch)
            plsc.subcore_barrier()
            pltpu.sync_copy(acc_sh.at[pl.ds(s * rpt, rpt)],
                            o_hbm.at[pl.ds(s * rpt, rpt)])

        @pl.when(c == 0)
        def _():
            run(xlo_hbm, olo_hbm)

        @pl.when(c == 1)
        def _():
            run(xhi_hbm, ohi_hbm)

    return agg_kernel(x_lo, x_hi, src, dst)


def _mlp_tc(a_lo, a_hi, w1a, w1b, b1, w2, b2, sum_inputs, split_out):
    n = a_lo.shape[0]
    kh = a_lo.shape[1]
    h = w2.shape[1]
    blk = 1264
    hiprec = lax.Precision.HIGHEST

    def body(alo_ref, ahi_ref, w1a_ref, w1b_ref, b1_ref, w2_ref, b2_ref,
             *out_refs):
        if sum_inputs:
            t = jnp.dot(alo_ref[...] + ahi_ref[...], w1a_ref[...],
                        preferred_element_type=jnp.float32, precision=hiprec)
        else:
            t = jnp.dot(alo_ref[...], w1a_ref[...],
                        preferred_element_type=jnp.float32, precision=hiprec)
            t += jnp.dot(ahi_ref[...], w1b_ref[...],
                         preferred_element_type=jnp.float32, precision=hiprec)
        t = jnp.maximum(t + b1_ref[...], 0.0)
        o = jnp.dot(t, w2_ref[...],
                    preferred_element_type=jnp.float32, precision=hiprec)
        o = jnp.maximum(o + b2_ref[...], 0.0)
        if split_out:
            out_refs[0][...] = o[:, : h // 2]
            out_refs[1][...] = o[:, h // 2:]
        else:
            out_refs[0][...] = o

    if split_out:
        out_shape = (
            jax.ShapeDtypeStruct((n, h // 2), jnp.float32),
            jax.ShapeDtypeStruct((n, h // 2), jnp.float32),
        )
        out_specs = (
            pl.BlockSpec((blk, h // 2), lambda i: (i, 0)),
            pl.BlockSpec((blk, h // 2), lambda i: (i, 0)),
        )
    else:
        out_shape = jax.ShapeDtypeStruct((n, h), jnp.float32)
        out_specs = pl.BlockSpec((blk, h), lambda i: (i, 0))

    return pl.pallas_call(
        body,
        grid=(n // blk,),
        in_specs=[
            pl.BlockSpec((blk, kh), lambda i: (i, 0)),
            pl.BlockSpec((blk, kh), lambda i: (i, 0)),
            pl.BlockSpec(w1a.shape, lambda i: (0, 0)),
            pl.BlockSpec(w1b.shape, lambda i: (0, 0)),
            pl.BlockSpec((1, h), lambda i: (0, 0)),
            pl.BlockSpec((h, h), lambda i: (0, 0)),
            pl.BlockSpec((1, h), lambda i: (0, 0)),
        ],
        out_specs=out_specs,
        out_shape=out_shape,
    )(a_lo, a_hi, w1a, w1b, b1, w2, b2)


def kernel(x, edge_index, W0_1, b0_1, g0, be0, W0_2, b0_2,
           W1_1, b1_1, g1, be1, W1_2, b1_2):
    n, d = x.shape
    h = W0_1.shape[1]
    e = edge_index.shape[1]

    src = edge_index[0].astype(jnp.int32)
    dst = edge_index[1].astype(jnp.int32)

    blk_e = 2 * _TILES * _CH * 4
    e_pad = -(-e // blk_e) * blk_e
    npad = -(-n // (_TILES * 8)) * (_TILES * 8)
    if e_pad != e:
        src = jnp.concatenate([src, jnp.zeros((e_pad - e,), jnp.int32)])
        dst = jnp.concatenate([dst, jnp.full((e_pad - e,), n, jnp.int32)])
    src = jnp.pad(src.reshape(e_pad // _CH, _CH), ((0, 2), (0, 0)))
    dst = jnp.pad(dst.reshape(e_pad // _CH, _CH), ((0, 2), (0, 0)))

    s0 = g0 / jnp.sqrt(1.0 + BN_EPS_)
    w0s = W0_1 * s0[None, :]
    b0f = (b0_1 * s0 + be0).reshape(1, h)
    s1 = g1 / jnp.sqrt(1.0 + BN_EPS_)
    w1s = W1_1 * s1[None, :]
    b1f = (b1_1 * s1 + be1).reshape(1, h)
    b0_2r = b0_2.reshape(1, h)
    b1_2r = b1_2.reshape(1, h)

    xp = jnp.pad(x, ((0, npad - n), (0, 0)))
    zp = jnp.zeros_like(xp)
    p0, p1 = _agg_edge_split(xp, zp, src, dst)
    h_lo, h_hi = _mlp_tc(p0, p1, w0s, w0s, b0f, W0_2, b0_2r,
                         sum_inputs=True, split_out=True)

    a1_lo, a1_hi = _agg_feat_split(h_lo, h_hi, src, dst)
    out = _mlp_tc(a1_lo, a1_hi, w1s[: h // 2], w1s[h // 2:],
                  b1f, W1_2, b1_2r, sum_inputs=False, split_out=False)
    return out[:n]
